# Initial kernel scaffold; baseline (speedup 1.0000x reference)
import jax
import jax.numpy as jnp
from jax.experimental import pallas as pl


def kernel(x, edge_index):
    raise NotImplementedError("write your pallas kernel here")


# speedup vs baseline: 4.5866x; 4.5866x over previous
import functools

import jax
import jax.numpy as jnp
from jax import lax
from jax.experimental import pallas as pl
from jax.experimental.pallas import tpu as pltpu
from jax.experimental.pallas import tpu_sc as plsc

N_NODES = 10000
D_FEAT = 128

_NC = 2
_NS = 16
_NW = _NC * _NS
_C = 128
_RPT = 640
_R_PAD = _RPT * _NS


def _sc_gather_scatter(x, src, dst, n_chunks):
    mesh = plsc.VectorSubcoreMesh(core_axis_name="c", subcore_axis_name="s")

    @functools.partial(
        pl.kernel,
        out_type=jax.ShapeDtypeStruct((_NC, _R_PAD, D_FEAT), jnp.float32),
        mesh=mesh,
        scratch_types=[
            pltpu.VMEM((_C,), jnp.int32),
            pltpu.VMEM((_C,), jnp.int32),
            pltpu.VMEM((_C, D_FEAT), jnp.float32),
            pltpu.VMEM((16, D_FEAT), jnp.float32),
            pltpu.VMEM_SHARED((_R_PAD, D_FEAT), jnp.float32),
            pltpu.SemaphoreType.DMA,
        ],
    )
    def k(x_hbm, src_hbm, dst_hbm, out_hbm, src_v, dst_v, rows_v, zer_v, acc, sem):
        c = lax.axis_index("c")
        s = lax.axis_index("s")
        wid = c * _NS + s
        epw = n_chunks * _C

        def zinit(i, carry):
            zer_v[i // 8, pl.ds((i % 8) * 16, 16)] = jnp.zeros((16,), jnp.float32)
            return carry

        lax.fori_loop(0, 16 * 8, zinit, 0)

        def zacc(i, carry):
            pltpu.sync_copy(zer_v, acc.at[pl.ds(s * _RPT + i * 16, 16)])
            return carry

        lax.fori_loop(0, _RPT // 16, zacc, 0)
        plsc.subcore_barrier()

        base_w = wid * epw

        def step(g, carry):
            b = base_w + g * _C
            pltpu.sync_copy(src_hbm.at[pl.ds(b, _C)], src_v)
            gather = pltpu.async_copy(x_hbm.at[src_v], rows_v, sem)
            pltpu.sync_copy(dst_hbm.at[pl.ds(b, _C)], dst_v)
            gather.wait()
            pltpu.sync_copy(rows_v, acc.at[dst_v], add=True)
            return carry

        lax.fori_loop(0, n_chunks, step, 0)

        plsc.subcore_barrier()
        pltpu.sync_copy(
            acc.at[pl.ds(s * _RPT, _RPT)],
            out_hbm.at[c, pl.ds(s * _RPT, _RPT)],
        )

    return k(x, src, dst)


def _combine(p0, p1):
    def body(a_ref, b_ref, o_ref):
        o_ref[...] = a_ref[...] + b_ref[...]

    return pl.pallas_call(
        body,
        out_shape=jax.ShapeDtypeStruct((N_NODES, D_FEAT), jnp.float32),
        grid=(10,),
        in_specs=[
            pl.BlockSpec((N_NODES // 10, D_FEAT), lambda i: (i, 0)),
            pl.BlockSpec((N_NODES // 10, D_FEAT), lambda i: (i, 0)),
        ],
        out_specs=pl.BlockSpec((N_NODES // 10, D_FEAT), lambda i: (i, 0)),
    )(p0, p1)


def kernel(x, edge_index):
    n_edges = edge_index.shape[1]
    src = edge_index[0].astype(jnp.int32)
    dst = edge_index[1].astype(jnp.int32)

    chunk = _NW * _C
    n_chunks = -(-n_edges // chunk)
    e_pad = n_chunks * chunk
    pad = e_pad - n_edges
    if pad:
        src = jnp.concatenate([src, jnp.zeros((pad,), jnp.int32)])
        dst = jnp.concatenate([dst, jnp.full((pad,), N_NODES, jnp.int32)])

    partials = _sc_gather_scatter(x, src, dst, n_chunks)
    return _combine(partials[0, :N_NODES], partials[1, :N_NODES])
